# Initial kernel scaffold; baseline (speedup 1.0000x reference)
import jax
import jax.numpy as jnp
from jax.experimental import pallas as pl


def kernel(pos, edge_index, batch, w_pool, W1, b1, g1, be1, W2, b2, g2, be2, W3, b3, g3, be3, W4, b4, g4, be4, W5, b5, g5, be5, fW1, fb1, fW2, fb2, fW3, fb3):
    raise NotImplementedError("write your pallas kernel here")


# speedup vs baseline: 533.3238x; 533.3238x over previous
import jax
import jax.numpy as jnp
from jax import lax
from jax.experimental import pallas as pl
from jax.experimental.pallas import tpu as pltpu
from jax.experimental.pallas import tpu_sc as plsc

_N = 100000
_K = 1024
_L = 16
_NC = 2
_NS = 16
_NT = _NC * _NS
_CHUNK = 4000
_SENT = 0x40000000
_NW = (_N // 32 + 1 + _L - 1) // _L * _L


def _rtne_bf16(x):
    u = lax.bitcast_convert_type(x, jnp.int32)
    r = (u + 0x7FFF + (lax.shift_right_logical(u, 16) & 1)) & jnp.int32(-65536)
    return lax.bitcast_convert_type(r, jnp.float32)


def _popcount(v):
    v = v - ((v >> 1) & 0x55555555)
    v = (v & 0x33333333) + ((v >> 2) & 0x33333333)
    v = (v + (v >> 4)) & 0x0F0F0F0F
    return lax.shift_right_logical(v * 0x01010101, 24)


def _sc_edge_filter(src_hbm, dst_hbm, words_hbm, pref_hbm, codes_out, cnt_out,
                    bw_v, pf_v, src_v, dst_v, grp_v, cnt_v):
    c = lax.axis_index("c")
    s = lax.axis_index("s")
    wid = s * _NC + c

    pltpu.sync_copy(words_hbm, bw_v)
    pltpu.sync_copy(pref_hbm, pf_v)

    ept = src_hbm.shape[0] // _NT
    base0 = wid * ept
    nchunks = ept // _CHUNK
    rem = ept % _CHUNK

    def _rank(idx, word):
        low = word & ((1 << (idx & 31)) - 1)
        base = plsc.load_gather(pf_v, [lax.shift_right_logical(idx, 5)])
        return base + _popcount(low)

    def _groups(hi, ng0):
        def _grp(gi, ng):
            s_ids = src_v[pl.ds(gi * _L, _L)]
            d_ids = dst_v[pl.ds(gi * _L, _L)]
            s_w = plsc.load_gather(bw_v, [lax.shift_right_logical(s_ids, 5)])
            d_w = plsc.load_gather(bw_v, [lax.shift_right_logical(d_ids, 5)])
            s_hit = lax.shift_right_logical(s_w, s_ids & 31) & 1
            d_hit = lax.shift_right_logical(d_w, d_ids & 31) & 1
            m = (s_hit & d_hit) == 1
            hit = jnp.any(m)

            @pl.when(hit)
            def _emit():
                code = _rank(d_ids, d_w) * _K + _rank(s_ids, s_w)
                grp_v[...] = jnp.where(m, code, _SENT)
                pltpu.sync_copy(grp_v, codes_out.at[pl.ds(base0 + ng * _L, _L)])
            return ng + jnp.where(hit, 1, 0)
        return lax.fori_loop(0, hi, _grp, ng0)

    def _chunk(ci, ng):
        cbase = base0 + ci * _CHUNK
        pltpu.sync_copy(src_hbm.at[pl.ds(cbase, _CHUNK)], src_v)
        pltpu.sync_copy(dst_hbm.at[pl.ds(cbase, _CHUNK)], dst_v)
        return _groups(_CHUNK // _L, ng)
    ngroups = lax.fori_loop(0, nchunks, _chunk, jnp.int32(0))

    if rem:
        rbase = base0 + nchunks * _CHUNK
        pltpu.sync_copy(src_hbm.at[pl.ds(rbase, rem)], src_v.at[pl.ds(0, rem)])
        pltpu.sync_copy(dst_hbm.at[pl.ds(rbase, rem)], dst_v.at[pl.ds(0, rem)])
        ngroups = _groups(rem // _L, ngroups)

    cnt_v[...] = jnp.broadcast_to(ngroups, (_L,))
    pltpu.sync_copy(cnt_v, cnt_out.at[pl.ds(wid * _L, _L)])


def _sc_count_build(codes_hbm, cnt_hbm, c_out, cnt_v, code_v, cl_v):
    c = lax.axis_index("c")
    s = lax.axis_index("s")
    wid = s * _NC + c
    rows = _K // _NT
    lo = wid * rows

    zeros = jnp.zeros((_L,), jnp.float32)
    ones = jnp.ones((_L,), jnp.float32)

    for r in range(rows):
        def _z(i, carry, r=r):
            cl_v[r, pl.ds(i * _L, _L)] = zeros
            return carry
        lax.fori_loop(0, _K // _L, _z, 0)

    pltpu.sync_copy(cnt_hbm, cnt_v)

    ept = codes_hbm.shape[0] // _NT
    cb = _CHUNK
    gpc = cb // _L

    for t in range(_NT):
        g_t = jnp.max(cnt_v[pl.ds(t * _L, _L)])
        nch = (g_t + gpc - 1) // gpc

        def _ch(ci, carry, t=t, g_t=g_t):
            pltpu.sync_copy(codes_hbm.at[pl.ds(t * ept + ci * cb, cb)], code_v)
            ghere = jnp.minimum(gpc, g_t - ci * gpc)

            def _g(gi, carry2):
                code = code_v[pl.ds(gi * _L, _L)]
                d2 = lax.shift_right_logical(code, 10)
                s2 = code & (_K - 1)
                m = (d2 >= lo) & (d2 < lo + rows)
                r = jnp.where(m, d2 - lo, 0)
                plsc.addupdate_scatter(cl_v, [r, s2], ones, mask=m)
                return carry2
            lax.fori_loop(0, ghere, _g, 0)
            return carry
        lax.fori_loop(0, nch, _ch, 0)

    pltpu.sync_copy(cl_v, c_out.at[pl.ds(lo, rows)])


def _tc_score(pos3_ref, wn_ref, out_ref):
    c0 = _rtne_bf16(pos3_ref[0, :, :])
    c1 = _rtne_bf16(pos3_ref[1, :, :])
    c2 = _rtne_bf16(pos3_ref[2, :, :])
    out_ref[...] = (c0 * wn_ref[0] + c1 * wn_ref[1] + c2 * wn_ref[2]) / wn_ref[3]


def _tc_dense(xp_ref, ss_ref, cs_ref,
              w1, b1, g1, be1, w2, b2, g2, be2, w3, b3, g3, be3,
              w4, b4, g4, be4, w5, b5, g5, be5,
              fw1, fb1, fw2, fb2, fw3, fb3, out_ref):
    C = cs_ref[...]
    deg = 1.0 + jnp.sum(C, axis=1, keepdims=True)
    di = lax.rsqrt(deg)
    dii = di * di

    mm = lambda a, bb: jnp.dot(_rtne_bf16(a), _rtne_bf16(bb[...]),
                               preferred_element_type=jnp.float32,
                               precision=lax.Precision.HIGHEST)
    x = xp_ref[...] * ss_ref[...]
    for (w, b, g, be) in ((w1, b1, g1, be1), (w2, b2, g2, be2),
                          (w3, b3, g3, be3), (w4, b4, g4, be4),
                          (w5, b5, g5, be5)):
        h = mm(x, w)
        agg = di * jnp.dot(C, di * h, preferred_element_type=jnp.float32,
                           precision=lax.Precision.HIGHEST)
        out = agg + dii * h + b[...]
        mu = out.mean(axis=0, keepdims=True)
        dlt = out - mu
        var = (dlt * dlt).mean(axis=0, keepdims=True)
        x = jnp.maximum(dlt / jnp.sqrt(var + 1e-5) * g[...] + be[...], 0.0)

    pooled = jnp.sum(x, axis=0, keepdims=True)
    h = jnp.maximum(mm(pooled, fw1) + fb1[...], 0.0)
    h = jnp.maximum(mm(h, fw2) + fb2[...], 0.0)
    h = mm(h, fw3) + fb3[...]
    z = h - jnp.max(h, axis=-1, keepdims=True)
    out_ref[...] = z - jnp.log(jnp.sum(jnp.exp(z), axis=-1, keepdims=True))


def kernel(pos, edge_index, batch, w_pool, W1, b1, g1, be1, W2, b2, g2, be2,
           W3, b3, g3, be3, W4, b4, g4, be4, W5, b5, g5, be5,
           fW1, fb1, fW2, fb2, fW3, fb3):
    pos = pos.astype(jnp.float32)
    wu = lax.bitcast_convert_type(w_pool.astype(jnp.float32), jnp.int32)
    wu = (wu + 0x7FFF + (lax.shift_right_logical(wu, 16) & 1)) & jnp.int32(-65536)
    wb = lax.bitcast_convert_type(wu, jnp.float32)
    wn = jnp.concatenate(
        [wb, jnp.linalg.norm(w_pool)[None]]).astype(jnp.float32)

    rows, cols = 800, 125
    pos3 = jnp.transpose(pos).reshape(3, rows, cols)
    score2 = pl.pallas_call(
        _tc_score,
        out_shape=jax.ShapeDtypeStruct((rows, cols), jnp.float32),
        in_specs=[pl.BlockSpec(memory_space=pltpu.VMEM),
                  pl.BlockSpec(memory_space=pltpu.SMEM)],
        out_specs=pl.BlockSpec(memory_space=pltpu.VMEM),
    )(pos3, wn)
    score = jnp.tanh(score2.reshape(_N))

    perm = lax.top_k(score, _K)[1].astype(jnp.int32)
    kept = jnp.sort(perm)
    xp = jnp.take(pos, kept, axis=0)
    ss = jnp.take(score, kept)[:, None]

    member = jnp.zeros((_NW * 32,), jnp.int32).at[kept].set(1)
    member = member.reshape(_NW, 32)
    words = jnp.sum(
        member * (jnp.int32(1) << jnp.arange(32, dtype=jnp.int32)), axis=1,
        dtype=jnp.int32)
    pops = jnp.sum(member, axis=1, dtype=jnp.int32)
    pref = jnp.cumsum(pops) - pops

    src = edge_index[0].astype(jnp.int32)
    dst = edge_index[1].astype(jnp.int32)
    e = src.shape[0]
    ept = -(-e // _NT)
    ept = -(-ept // _CHUNK) * _CHUNK
    epad = ept * _NT
    if epad != e:
        src = jnp.pad(src, (0, epad - e), constant_values=_N)
        dst = jnp.pad(dst, (0, epad - e), constant_values=_N)

    mesh = plsc.VectorSubcoreMesh(core_axis_name="c", subcore_axis_name="s")
    codes, cnts = pl.kernel(
        _sc_edge_filter,
        mesh=mesh,
        compiler_params=pltpu.CompilerParams(needs_layout_passes=False),
        out_type=(jax.ShapeDtypeStruct((epad,), jnp.int32),
                  jax.ShapeDtypeStruct((_NT * _L,), jnp.int32)),
        scratch_types=[
            pltpu.VMEM((_NW,), jnp.int32),
            pltpu.VMEM((_NW,), jnp.int32),
            pltpu.VMEM((_CHUNK,), jnp.int32),
            pltpu.VMEM((_CHUNK,), jnp.int32),
            pltpu.VMEM((_L,), jnp.int32),
            pltpu.VMEM((_L,), jnp.int32),
        ],
    )(src, dst, words, pref)

    cs = pl.kernel(
        _sc_count_build,
        mesh=mesh,
        compiler_params=pltpu.CompilerParams(needs_layout_passes=False),
        out_type=jax.ShapeDtypeStruct((_K, _K), jnp.float32),
        scratch_types=[
            pltpu.VMEM((_NT * _L,), jnp.int32),
            pltpu.VMEM((_CHUNK,), jnp.int32),
            pltpu.VMEM((_K // _NT, _K), jnp.float32),
        ],
    )(codes, cnts)

    row = lambda v: v.reshape(1, -1).astype(jnp.float32)
    out = pl.pallas_call(
        _tc_dense,
        out_shape=jax.ShapeDtypeStruct((1, 100), jnp.float32),
    )(xp, ss, cs,
      W1, row(b1), row(g1), row(be1), W2, row(b2), row(g2), row(be2),
      W3, row(b3), row(g3), row(be3), W4, row(b4), row(g4), row(be4),
      W5, row(b5), row(g5), row(be5),
      fW1, row(fb1), fW2, row(fb2), fW3, row(fb3))
    return out


# speedup vs baseline: 697.5215x; 1.3079x over previous
import jax
import jax.numpy as jnp
from jax import lax
from jax.experimental import pallas as pl
from jax.experimental.pallas import tpu as pltpu
from jax.experimental.pallas import tpu_sc as plsc

_N = 100000
_K = 1024
_L = 16
_NC = 2
_NS = 16
_NT = _NC * _NS
_CHUNK = 4096
_U = 4
_SENT = 0x40000000
_NW = (_N // 32 + 1 + _L - 1) // _L * _L


def _rtne_bf16(x):
    u = lax.bitcast_convert_type(x, jnp.int32)
    r = (u + 0x7FFF + (lax.shift_right_logical(u, 16) & 1)) & jnp.int32(-65536)
    return lax.bitcast_convert_type(r, jnp.float32)


def _popcount(v):
    v = v - ((v >> 1) & 0x55555555)
    v = (v & 0x33333333) + ((v >> 2) & 0x33333333)
    v = (v + (v >> 4)) & 0x0F0F0F0F
    return lax.shift_right_logical(v * 0x01010101, 24)


def _sc_edge_filter(src_hbm, dst_hbm, words_hbm, pref_hbm, codes_out, cnt_out,
                    bw_v, pf_v, src_v, dst_v, grp_v, cnt_v):
    c = lax.axis_index("c")
    s = lax.axis_index("s")
    wid = s * _NC + c

    pltpu.sync_copy(words_hbm, bw_v)
    pltpu.sync_copy(pref_hbm, pf_v)

    ept = src_hbm.shape[0] // _NT
    base0 = wid * ept
    nchunks = ept // _CHUNK

    def _rank(idx, word):
        low = word & ((1 << (idx & 31)) - 1)
        base = plsc.load_gather(pf_v, [lax.shift_right_logical(idx, 5)])
        return base + _popcount(low)

    def _groups(hi, ng0):
        def _grp(gi, ng):
            tests = []
            for u in range(_U):
                off = (gi * _U + u) * _L
                s_ids = src_v[pl.ds(off, _L)]
                d_ids = dst_v[pl.ds(off, _L)]
                s_w = plsc.load_gather(bw_v,
                                       [lax.shift_right_logical(s_ids, 5)])
                d_w = plsc.load_gather(bw_v,
                                       [lax.shift_right_logical(d_ids, 5)])
                hb = (lax.shift_right_logical(s_w, s_ids & 31)
                      & lax.shift_right_logical(d_w, d_ids & 31) & 1)
                m = hb == 1
                tests.append((s_ids, d_ids, s_w, d_w, m, jnp.any(m)))
            for (s_ids, d_ids, s_w, d_w, m, hit) in tests:
                @pl.when(hit)
                def _emit(s_ids=s_ids, d_ids=d_ids, s_w=s_w, d_w=d_w,
                          m=m, ng=ng):
                    code = _rank(d_ids, d_w) * _K + _rank(s_ids, s_w)
                    grp_v[...] = jnp.where(m, code, _SENT)
                    pltpu.sync_copy(
                        grp_v, codes_out.at[pl.ds(base0 + ng * _L, _L)])
                ng = ng + jnp.where(hit, 1, 0)
            return ng
        return lax.fori_loop(0, hi // _U, _grp, ng0)

    def _chunk(ci, ng):
        cbase = base0 + ci * _CHUNK
        pltpu.sync_copy(src_hbm.at[pl.ds(cbase, _CHUNK)], src_v)
        pltpu.sync_copy(dst_hbm.at[pl.ds(cbase, _CHUNK)], dst_v)
        return _groups(_CHUNK // _L, ng)
    ngroups = lax.fori_loop(0, nchunks, _chunk, jnp.int32(0))

    cnt_v[...] = jnp.broadcast_to(ngroups, (_L,))
    pltpu.sync_copy(cnt_v, cnt_out.at[pl.ds(wid * _L, _L)])


def _sc_count_build(codes_hbm, cnt_hbm, c_out, cnt_v, code_v, cl_v):
    c = lax.axis_index("c")
    s = lax.axis_index("s")
    wid = s * _NC + c
    rows = _K // _NT
    lo = wid * rows

    zeros = jnp.zeros((_L,), jnp.float32)
    ones = jnp.ones((_L,), jnp.float32)

    for r in range(rows):
        def _z(i, carry, r=r):
            cl_v[r, pl.ds(i * _L, _L)] = zeros
            return carry
        lax.fori_loop(0, _K // _L, _z, 0)

    pltpu.sync_copy(cnt_hbm, cnt_v)

    ept = codes_hbm.shape[0] // _NT
    cb = _CHUNK
    gpc = cb // _L

    for t in range(_NT):
        g_t = jnp.max(cnt_v[pl.ds(t * _L, _L)])
        nch = (g_t + gpc - 1) // gpc

        def _ch(ci, carry, t=t, g_t=g_t):
            pltpu.sync_copy(codes_hbm.at[pl.ds(t * ept + ci * cb, cb)], code_v)
            ghere = jnp.minimum(gpc, g_t - ci * gpc)

            def _g(gi, carry2):
                code = code_v[pl.ds(gi * _L, _L)]
                d2 = lax.shift_right_logical(code, 10)
                s2 = code & (_K - 1)
                m = (d2 >= lo) & (d2 < lo + rows)
                r = jnp.where(m, d2 - lo, 0)
                plsc.addupdate_scatter(cl_v, [r, s2], ones, mask=m)
                return carry2
            lax.fori_loop(0, ghere, _g, 0)
            return carry
        lax.fori_loop(0, nch, _ch, 0)

    pltpu.sync_copy(cl_v, c_out.at[pl.ds(lo, rows)])


def _tc_score(pos3_ref, wn_ref, out_ref):
    c0 = _rtne_bf16(pos3_ref[0, :, :])
    c1 = _rtne_bf16(pos3_ref[1, :, :])
    c2 = _rtne_bf16(pos3_ref[2, :, :])
    out_ref[...] = (c0 * wn_ref[0] + c1 * wn_ref[1] + c2 * wn_ref[2]) / wn_ref[3]


def _tc_dense(xp_ref, ss_ref, cs_ref,
              w1, b1, g1, be1, w2, b2, g2, be2, w3, b3, g3, be3,
              w4, b4, g4, be4, w5, b5, g5, be5,
              fw1, fb1, fw2, fb2, fw3, fb3, out_ref):
    C = cs_ref[...]
    deg = 1.0 + jnp.sum(C, axis=1, keepdims=True)
    di = lax.rsqrt(deg)
    dii = di * di

    mm = lambda a, bb: jnp.dot(_rtne_bf16(a), _rtne_bf16(bb[...]),
                               preferred_element_type=jnp.float32,
                               precision=lax.Precision.HIGHEST)
    x = xp_ref[...] * ss_ref[...]
    for (w, b, g, be) in ((w1, b1, g1, be1), (w2, b2, g2, be2),
                          (w3, b3, g3, be3), (w4, b4, g4, be4),
                          (w5, b5, g5, be5)):
        h = mm(x, w)
        agg = di * jnp.dot(C, di * h, preferred_element_type=jnp.float32,
                           precision=lax.Precision.HIGHEST)
        out = agg + dii * h + b[...]
        mu = out.mean(axis=0, keepdims=True)
        dlt = out - mu
        var = (dlt * dlt).mean(axis=0, keepdims=True)
        x = jnp.maximum(dlt / jnp.sqrt(var + 1e-5) * g[...] + be[...], 0.0)

    pooled = jnp.sum(x, axis=0, keepdims=True)
    h = jnp.maximum(mm(pooled, fw1) + fb1[...], 0.0)
    h = jnp.maximum(mm(h, fw2) + fb2[...], 0.0)
    h = mm(h, fw3) + fb3[...]
    z = h - jnp.max(h, axis=-1, keepdims=True)
    out_ref[...] = z - jnp.log(jnp.sum(jnp.exp(z), axis=-1, keepdims=True))


def kernel(pos, edge_index, batch, w_pool, W1, b1, g1, be1, W2, b2, g2, be2,
           W3, b3, g3, be3, W4, b4, g4, be4, W5, b5, g5, be5,
           fW1, fb1, fW2, fb2, fW3, fb3):
    pos = pos.astype(jnp.float32)
    wu = lax.bitcast_convert_type(w_pool.astype(jnp.float32), jnp.int32)
    wu = (wu + 0x7FFF + (lax.shift_right_logical(wu, 16) & 1)) & jnp.int32(-65536)
    wb = lax.bitcast_convert_type(wu, jnp.float32)
    wn = jnp.concatenate(
        [wb, jnp.linalg.norm(w_pool)[None]]).astype(jnp.float32)

    rows, cols = 800, 125
    pos3 = jnp.transpose(pos).reshape(3, rows, cols)
    score2 = pl.pallas_call(
        _tc_score,
        out_shape=jax.ShapeDtypeStruct((rows, cols), jnp.float32),
        in_specs=[pl.BlockSpec(memory_space=pltpu.VMEM),
                  pl.BlockSpec(memory_space=pltpu.SMEM)],
        out_specs=pl.BlockSpec(memory_space=pltpu.VMEM),
    )(pos3, wn)
    score = jnp.tanh(score2.reshape(_N))

    perm = lax.top_k(score, _K)[1].astype(jnp.int32)
    kept = jnp.sort(perm)
    xp = jnp.take(pos, kept, axis=0)
    ss = jnp.take(score, kept)[:, None]

    member = jnp.zeros((_NW * 32,), jnp.int32).at[kept].set(1)
    member = member.reshape(_NW, 32)
    words = jnp.sum(
        member * (jnp.int32(1) << jnp.arange(32, dtype=jnp.int32)), axis=1,
        dtype=jnp.int32)
    pops = jnp.sum(member, axis=1, dtype=jnp.int32)
    pref = jnp.cumsum(pops) - pops

    src = edge_index[0].astype(jnp.int32)
    dst = edge_index[1].astype(jnp.int32)
    e = src.shape[0]
    ept = -(-e // _NT)
    ept = -(-ept // _CHUNK) * _CHUNK
    epad = ept * _NT
    if epad != e:
        src = jnp.pad(src, (0, epad - e), constant_values=_N)
        dst = jnp.pad(dst, (0, epad - e), constant_values=_N)

    mesh = plsc.VectorSubcoreMesh(core_axis_name="c", subcore_axis_name="s")
    codes, cnts = pl.kernel(
        _sc_edge_filter,
        mesh=mesh,
        compiler_params=pltpu.CompilerParams(needs_layout_passes=False),
        out_type=(jax.ShapeDtypeStruct((epad,), jnp.int32),
                  jax.ShapeDtypeStruct((_NT * _L,), jnp.int32)),
        scratch_types=[
            pltpu.VMEM((_NW,), jnp.int32),
            pltpu.VMEM((_NW,), jnp.int32),
            pltpu.VMEM((_CHUNK,), jnp.int32),
            pltpu.VMEM((_CHUNK,), jnp.int32),
            pltpu.VMEM((_L,), jnp.int32),
            pltpu.VMEM((_L,), jnp.int32),
        ],
    )(src, dst, words, pref)

    cs = pl.kernel(
        _sc_count_build,
        mesh=mesh,
        compiler_params=pltpu.CompilerParams(needs_layout_passes=False),
        out_type=jax.ShapeDtypeStruct((_K, _K), jnp.float32),
        scratch_types=[
            pltpu.VMEM((_NT * _L,), jnp.int32),
            pltpu.VMEM((_CHUNK,), jnp.int32),
            pltpu.VMEM((_K // _NT, _K), jnp.float32),
        ],
    )(codes, cnts)

    row = lambda v: v.reshape(1, -1).astype(jnp.float32)
    out = pl.pallas_call(
        _tc_dense,
        out_shape=jax.ShapeDtypeStruct((1, 100), jnp.float32),
    )(xp, ss, cs,
      W1, row(b1), row(g1), row(be1), W2, row(b2), row(g2), row(be2),
      W3, row(b3), row(g3), row(be3), W4, row(b4), row(g4), row(be4),
      W5, row(b5), row(g5), row(be5),
      fW1, row(fb1), fW2, row(fb2), fW3, row(fb3))
    return out


# speedup vs baseline: 794.5969x; 1.1392x over previous
import jax
import jax.numpy as jnp
from jax import lax
from jax.experimental import pallas as pl
from jax.experimental.pallas import tpu as pltpu
from jax.experimental.pallas import tpu_sc as plsc

_N = 100000
_K = 1024
_L = 16
_NC = 2
_NS = 16
_NT = _NC * _NS
_CHUNK = 4096
_U = 8
_SENT = 0x40000000
_NW = (_N // 32 + 1 + _L - 1) // _L * _L


def _rtne_bf16(x):
    u = lax.bitcast_convert_type(x, jnp.int32)
    r = (u + 0x7FFF + (lax.shift_right_logical(u, 16) & 1)) & jnp.int32(-65536)
    return lax.bitcast_convert_type(r, jnp.float32)


def _pick_cb(ept):
    for cand in (4096, 2048, 1024, 512, 400, 256, 208, 128, 80, 48):
        if ept % cand == 0:
            return cand
    return _L


def _popcount(v):
    v = v - ((v >> 1) & 0x55555555)
    v = (v & 0x33333333) + ((v >> 2) & 0x33333333)
    v = (v + (v >> 4)) & 0x0F0F0F0F
    return lax.shift_right_logical(v * 0x01010101, 24)


def _sc_edge_filter(src_hbm, dst_hbm, words_hbm, pref_hbm, codes_out, cnt_out,
                    bw_v, pf_v, src_v, dst_v, grp_v, cnt_v):
    c = lax.axis_index("c")
    s = lax.axis_index("s")
    wid = s * _NC + c

    pltpu.sync_copy(words_hbm, bw_v)
    pltpu.sync_copy(pref_hbm, pf_v)

    ept = src_hbm.shape[0] // _NT
    base0 = wid * ept
    nchunks = ept // _CHUNK
    remn = ept % _CHUNK

    def _rank(idx, word):
        low = word & ((1 << (idx & 31)) - 1)
        base = plsc.load_gather(pf_v, [lax.shift_right_logical(idx, 5)])
        return base + _popcount(low)

    def _groups(hi, ng0):
        def _grp(gi, ng):
            tests = []
            for u in range(_U):
                off = (gi * _U + u) * _L
                s_ids = src_v[pl.ds(off, _L)]
                d_ids = dst_v[pl.ds(off, _L)]
                s_w = plsc.load_gather(bw_v,
                                       [lax.shift_right_logical(s_ids, 5)])
                d_w = plsc.load_gather(bw_v,
                                       [lax.shift_right_logical(d_ids, 5)])
                hb = (lax.shift_right_logical(s_w, s_ids & 31)
                      & lax.shift_right_logical(d_w, d_ids & 31) & 1)
                m = hb == 1
                tests.append((s_ids, d_ids, s_w, d_w, m, jnp.any(m)))
            for (s_ids, d_ids, s_w, d_w, m, hit) in tests:
                @pl.when(hit)
                def _emit(s_ids=s_ids, d_ids=d_ids, s_w=s_w, d_w=d_w,
                          m=m, ng=ng):
                    code = _rank(d_ids, d_w) * _K + _rank(s_ids, s_w)
                    grp_v[...] = jnp.where(m, code, _SENT)
                    pltpu.sync_copy(
                        grp_v, codes_out.at[pl.ds(base0 + ng * _L, _L)])
                ng = ng + jnp.where(hit, 1, 0)
            return ng
        return lax.fori_loop(0, hi // _U, _grp, ng0)

    def _groups1(n, off0, ng0):
        def _grp(gi, ng):
            s_ids = src_v[pl.ds((off0 + gi) * _L, _L)]
            d_ids = dst_v[pl.ds((off0 + gi) * _L, _L)]
            s_w = plsc.load_gather(bw_v, [lax.shift_right_logical(s_ids, 5)])
            d_w = plsc.load_gather(bw_v, [lax.shift_right_logical(d_ids, 5)])
            hb = (lax.shift_right_logical(s_w, s_ids & 31)
                  & lax.shift_right_logical(d_w, d_ids & 31) & 1)
            m = hb == 1

            @pl.when(jnp.any(m))
            def _emit():
                code = _rank(d_ids, d_w) * _K + _rank(s_ids, s_w)
                grp_v[...] = jnp.where(m, code, _SENT)
                pltpu.sync_copy(grp_v,
                                codes_out.at[pl.ds(base0 + ng * _L, _L)])
            return ng + jnp.where(jnp.any(m), 1, 0)
        return lax.fori_loop(0, n, _grp, ng0)

    def _chunk(ci, ng):
        cbase = base0 + ci * _CHUNK
        pltpu.sync_copy(src_hbm.at[pl.ds(cbase, _CHUNK)], src_v)
        pltpu.sync_copy(dst_hbm.at[pl.ds(cbase, _CHUNK)], dst_v)
        return _groups(_CHUNK // _L, ng)
    ngroups = lax.fori_loop(0, nchunks, _chunk, jnp.int32(0))

    if remn:
        rbase = base0 + nchunks * _CHUNK
        pltpu.sync_copy(src_hbm.at[pl.ds(rbase, remn)],
                        src_v.at[pl.ds(0, remn)])
        pltpu.sync_copy(dst_hbm.at[pl.ds(rbase, remn)],
                        dst_v.at[pl.ds(0, remn)])
        ng_grps = remn // _L
        ngroups = _groups(ng_grps - ng_grps % _U, ngroups)
        if ng_grps % _U:
            ngroups = _groups1(ng_grps % _U, ng_grps - ng_grps % _U, ngroups)

    cnt_v[...] = jnp.broadcast_to(ngroups, (_L,))
    pltpu.sync_copy(cnt_v, cnt_out.at[pl.ds(wid * _L, _L)])


def _sc_count_build(codes_hbm, cnt_hbm, c_out, cnt_v, code_v, cl_v):
    c = lax.axis_index("c")
    s = lax.axis_index("s")
    wid = s * _NC + c
    rows = _K // _NT
    lo = wid * rows

    zeros = jnp.zeros((_L,), jnp.float32)
    ones = jnp.ones((_L,), jnp.float32)

    for r in range(rows):
        def _z(i, carry, r=r):
            cl_v[r, pl.ds(i * _L, _L)] = zeros
            return carry
        lax.fori_loop(0, _K // _L, _z, 0)

    pltpu.sync_copy(cnt_hbm, cnt_v)

    ept = codes_hbm.shape[0] // _NT
    cb = _pick_cb(ept)
    gpc = cb // _L

    for t in range(_NT):
        g_t = jnp.max(cnt_v[pl.ds(t * _L, _L)])
        nch = (g_t + gpc - 1) // gpc

        def _ch(ci, carry, t=t, g_t=g_t):
            pltpu.sync_copy(codes_hbm.at[pl.ds(t * ept + ci * cb, cb)], code_v)
            ghere = jnp.minimum(gpc, g_t - ci * gpc)

            def _g(gi, carry2):
                code = code_v[pl.ds(gi * _L, _L)]
                d2 = lax.shift_right_logical(code, 10)
                s2 = code & (_K - 1)
                m = (d2 >= lo) & (d2 < lo + rows)
                r = jnp.where(m, d2 - lo, 0)
                plsc.addupdate_scatter(cl_v, [r, s2], ones, mask=m)
                return carry2
            lax.fori_loop(0, ghere, _g, 0)
            return carry
        lax.fori_loop(0, nch, _ch, 0)

    pltpu.sync_copy(cl_v, c_out.at[pl.ds(lo, rows)])


def _tc_score(pos3_ref, wn_ref, out_ref):
    c0 = _rtne_bf16(pos3_ref[0, :, :])
    c1 = _rtne_bf16(pos3_ref[1, :, :])
    c2 = _rtne_bf16(pos3_ref[2, :, :])
    out_ref[...] = (c0 * wn_ref[0] + c1 * wn_ref[1] + c2 * wn_ref[2]) / wn_ref[3]


def _tc_dense(xp_ref, ss_ref, cs_ref,
              w1, b1, g1, be1, w2, b2, g2, be2, w3, b3, g3, be3,
              w4, b4, g4, be4, w5, b5, g5, be5,
              fw1, fb1, fw2, fb2, fw3, fb3, out_ref):
    C = cs_ref[...]
    deg = 1.0 + jnp.sum(C, axis=1, keepdims=True)
    di = lax.rsqrt(deg)
    dii = di * di

    mm = lambda a, bb: jnp.dot(_rtne_bf16(a), _rtne_bf16(bb[...]),
                               preferred_element_type=jnp.float32,
                               precision=lax.Precision.HIGHEST)
    x = xp_ref[...] * ss_ref[...]
    for (w, b, g, be) in ((w1, b1, g1, be1), (w2, b2, g2, be2),
                          (w3, b3, g3, be3), (w4, b4, g4, be4),
                          (w5, b5, g5, be5)):
        h = mm(x, w)
        agg = di * jnp.dot(C, di * h, preferred_element_type=jnp.float32,
                           precision=lax.Precision.HIGHEST)
        out = agg + dii * h + b[...]
        mu = out.mean(axis=0, keepdims=True)
        dlt = out - mu
        var = (dlt * dlt).mean(axis=0, keepdims=True)
        x = jnp.maximum(dlt / jnp.sqrt(var + 1e-5) * g[...] + be[...], 0.0)

    pooled = jnp.sum(x, axis=0, keepdims=True)
    h = jnp.maximum(mm(pooled, fw1) + fb1[...], 0.0)
    h = jnp.maximum(mm(h, fw2) + fb2[...], 0.0)
    h = mm(h, fw3) + fb3[...]
    z = h - jnp.max(h, axis=-1, keepdims=True)
    out_ref[...] = z - jnp.log(jnp.sum(jnp.exp(z), axis=-1, keepdims=True))


def kernel(pos, edge_index, batch, w_pool, W1, b1, g1, be1, W2, b2, g2, be2,
           W3, b3, g3, be3, W4, b4, g4, be4, W5, b5, g5, be5,
           fW1, fb1, fW2, fb2, fW3, fb3):
    pos = pos.astype(jnp.float32)
    wu = lax.bitcast_convert_type(w_pool.astype(jnp.float32), jnp.int32)
    wu = (wu + 0x7FFF + (lax.shift_right_logical(wu, 16) & 1)) & jnp.int32(-65536)
    wb = lax.bitcast_convert_type(wu, jnp.float32)
    wn = jnp.concatenate(
        [wb, jnp.linalg.norm(w_pool)[None]]).astype(jnp.float32)

    rows, cols = 800, 125
    pos3 = jnp.transpose(pos).reshape(3, rows, cols)
    score2 = pl.pallas_call(
        _tc_score,
        out_shape=jax.ShapeDtypeStruct((rows, cols), jnp.float32),
        in_specs=[pl.BlockSpec(memory_space=pltpu.VMEM),
                  pl.BlockSpec(memory_space=pltpu.SMEM)],
        out_specs=pl.BlockSpec(memory_space=pltpu.VMEM),
    )(pos3, wn)
    score = jnp.tanh(score2.reshape(_N))

    perm = lax.top_k(score, _K)[1].astype(jnp.int32)
    kept = jnp.sort(perm)
    xp = jnp.take(pos, kept, axis=0)
    ss = jnp.take(score, kept)[:, None]

    member = jnp.zeros((_NW * 32,), jnp.int32).at[kept].set(1)
    member = member.reshape(_NW, 32)
    words = jnp.sum(
        member * (jnp.int32(1) << jnp.arange(32, dtype=jnp.int32)), axis=1,
        dtype=jnp.int32)
    pops = jnp.sum(member, axis=1, dtype=jnp.int32)
    pref = jnp.cumsum(pops) - pops

    src = edge_index[0].astype(jnp.int32)
    dst = edge_index[1].astype(jnp.int32)
    e = src.shape[0]
    if e % (_NT * _L) == 0:
        epad = e
    else:
        ept = -(-e // _NT)
        ept = -(-ept // _CHUNK) * _CHUNK
        epad = ept * _NT
        src = jnp.pad(src, (0, epad - e), constant_values=_N)
        dst = jnp.pad(dst, (0, epad - e), constant_values=_N)

    mesh = plsc.VectorSubcoreMesh(core_axis_name="c", subcore_axis_name="s")
    codes, cnts = pl.kernel(
        _sc_edge_filter,
        mesh=mesh,
        compiler_params=pltpu.CompilerParams(needs_layout_passes=False),
        out_type=(jax.ShapeDtypeStruct((epad,), jnp.int32),
                  jax.ShapeDtypeStruct((_NT * _L,), jnp.int32)),
        scratch_types=[
            pltpu.VMEM((_NW,), jnp.int32),
            pltpu.VMEM((_NW,), jnp.int32),
            pltpu.VMEM((_CHUNK,), jnp.int32),
            pltpu.VMEM((_CHUNK,), jnp.int32),
            pltpu.VMEM((_L,), jnp.int32),
            pltpu.VMEM((_L,), jnp.int32),
        ],
    )(src, dst, words, pref)

    cs = pl.kernel(
        _sc_count_build,
        mesh=mesh,
        compiler_params=pltpu.CompilerParams(needs_layout_passes=False),
        out_type=jax.ShapeDtypeStruct((_K, _K), jnp.float32),
        scratch_types=[
            pltpu.VMEM((_NT * _L,), jnp.int32),
            pltpu.VMEM((_pick_cb(epad // _NT),), jnp.int32),
            pltpu.VMEM((_K // _NT, _K), jnp.float32),
        ],
    )(codes, cnts)

    row = lambda v: v.reshape(1, -1).astype(jnp.float32)
    out = pl.pallas_call(
        _tc_dense,
        out_shape=jax.ShapeDtypeStruct((1, 100), jnp.float32),
    )(xp, ss, cs,
      W1, row(b1), row(g1), row(be1), W2, row(b2), row(g2), row(be2),
      W3, row(b3), row(g3), row(be3), W4, row(b4), row(g4), row(be4),
      W5, row(b5), row(g5), row(be5),
      fW1, row(fb1), fW2, row(fb2), fW3, row(fb3))
    return out
